# baseline (speedup 1.0000x reference)
import functools

import jax
import jax.numpy as jnp
from jax.experimental import pallas as pl
from jax.experimental.pallas import tpu as pltpu

T, D, E, DFF, DSH = 2048, 768, 64, 256, 512
TOPK = 2
NPAIR = T * TOPK
TB = 512
NTB = T // TB
BM = 64
NB = NPAIR // BM + E
NP = NB * BM


def _router_shared_body(x_ref, gate_w_ref, Sg_ref, Su_ref, Sd_ref, sgw_ref,
                        idx_ref, w_ref, sh_ref, xbf_ref):
    x = x_ref[...]
    logits = jnp.dot(x, gate_w_ref[...], preferred_element_type=jnp.float32)
    mx = jnp.max(logits, axis=-1, keepdims=True)
    p = jnp.exp(logits - mx)
    probs = p / jnp.sum(p, axis=-1, keepdims=True)
    iota_e = jax.lax.broadcasted_iota(jnp.int32, (TB, E), 1)
    m1 = jnp.max(probs, axis=-1, keepdims=True)
    i1 = jnp.min(jnp.where(probs == m1, iota_e, E), axis=-1)
    masked = jnp.where(iota_e == i1[:, None], -jnp.inf, probs)
    m2 = jnp.max(masked, axis=-1, keepdims=True)
    i2 = jnp.min(jnp.where(masked == m2, iota_e, E), axis=-1)
    denom = m1 + m2
    idx_ref[...] = jnp.concatenate([i1[:, None], i2[:, None]], axis=1)
    w_ref[...] = jnp.concatenate([m1 / denom, m2 / denom], axis=1)

    xb = x.astype(jnp.bfloat16)
    xbf_ref[...] = xb
    g = jnp.dot(xb, Sg_ref[...].astype(jnp.bfloat16),
                preferred_element_type=jnp.float32)
    u = jnp.dot(xb, Su_ref[...].astype(jnp.bfloat16),
                preferred_element_type=jnp.float32)
    h = (g * jax.nn.sigmoid(g) * u).astype(jnp.bfloat16)
    sh = jnp.dot(h, Sd_ref[...].astype(jnp.bfloat16),
                 preferred_element_type=jnp.float32)
    sgate = jax.nn.sigmoid(jnp.dot(x, sgw_ref[...],
                                   preferred_element_type=jnp.float32))
    sh_ref[...] = sgate * sh


def _router_shared(x, gate_w, Sg, Su, Sd, sgw):
    return pl.pallas_call(
        _router_shared_body,
        grid=(NTB,),
        in_specs=[
            pl.BlockSpec((TB, D), lambda t: (t, 0)),
            pl.BlockSpec((D, E), lambda t: (0, 0)),
            pl.BlockSpec((D, DSH), lambda t: (0, 0)),
            pl.BlockSpec((D, DSH), lambda t: (0, 0)),
            pl.BlockSpec((DSH, D), lambda t: (0, 0)),
            pl.BlockSpec((D, 1), lambda t: (0, 0)),
        ],
        out_specs=[
            pl.BlockSpec((TB, TOPK), lambda t: (t, 0)),
            pl.BlockSpec((TB, TOPK), lambda t: (t, 0)),
            pl.BlockSpec((TB, D), lambda t: (t, 0)),
            pl.BlockSpec((TB, D), lambda t: (t, 0)),
        ],
        out_shape=[
            jax.ShapeDtypeStruct((T, TOPK), jnp.int32),
            jax.ShapeDtypeStruct((T, TOPK), jnp.float32),
            jax.ShapeDtypeStruct((T, D), jnp.float32),
            jax.ShapeDtypeStruct((T, D), jnp.bfloat16),
        ],
        compiler_params=pltpu.CompilerParams(
            dimension_semantics=("arbitrary",)),
    )(x, gate_w, Sg, Su, Sd, sgw)


def _grouped_mlp_body(be_ref, x_ref, Wg_ref, Wu_ref, Wd_ref, w_ref, y_ref):
    xb = x_ref[...]
    g = jnp.dot(xb, Wg_ref[0].astype(jnp.bfloat16),
                preferred_element_type=jnp.float32)
    u = jnp.dot(xb, Wu_ref[0].astype(jnp.bfloat16),
                preferred_element_type=jnp.float32)
    h = (g * jax.nn.sigmoid(g) * u).astype(jnp.bfloat16)
    eo = jnp.dot(h, Wd_ref[0].astype(jnp.bfloat16),
                 preferred_element_type=jnp.float32)
    y_ref[...] = w_ref[...] * eo


def _grouped_mlp(block_expert, x_pad, Wg, Wu, Wd, w_pad):
    grid_spec = pltpu.PrefetchScalarGridSpec(
        num_scalar_prefetch=1,
        grid=(NB,),
        in_specs=[
            pl.BlockSpec((BM, D), lambda b, be: (b, 0)),
            pl.BlockSpec((1, D, DFF), lambda b, be: (be[b], 0, 0)),
            pl.BlockSpec((1, D, DFF), lambda b, be: (be[b], 0, 0)),
            pl.BlockSpec((1, DFF, D), lambda b, be: (be[b], 0, 0)),
            pl.BlockSpec((BM, 1), lambda b, be: (b, 0)),
        ],
        out_specs=pl.BlockSpec((BM, D), lambda b, be: (b, 0)),
    )
    return pl.pallas_call(
        _grouped_mlp_body,
        grid_spec=grid_spec,
        out_shape=jax.ShapeDtypeStruct((NP, D), jnp.float32),
        compiler_params=pltpu.CompilerParams(
            dimension_semantics=("arbitrary",)),
    )(block_expert, x_pad, Wg, Wu, Wd, w_pad)


def kernel(hidden_states, gate_w, Wg, Wu, Wd, Sg, Su, Sd, shared_gate_w):
    idx, w, sh, xbf = _router_shared(hidden_states, gate_w, Sg, Su, Sd,
                                     shared_gate_w)

    flat_e = idx.reshape(-1)
    order = jnp.argsort(flat_e, stable=True).astype(jnp.int32)
    counts = jnp.zeros((E,), jnp.int32).at[flat_e].add(1)
    nblk = (counts + BM - 1) // BM
    blk_end = jnp.cumsum(nblk)
    pad_off = (blk_end - nblk) * BM
    block_expert = jnp.minimum(
        jnp.searchsorted(blk_end, jnp.arange(NB, dtype=jnp.int32),
                         side="right").astype(jnp.int32), E - 1)
    grp_start = jnp.cumsum(counts) - counts
    e_sorted = flat_e[order]
    pp = (pad_off[e_sorted]
          + jnp.arange(NPAIR, dtype=jnp.int32) - grp_start[e_sorted])
    src = jnp.zeros((NP,), jnp.int32).at[pp].set(order // TOPK)
    w_pad = jnp.zeros((NP,), jnp.float32).at[pp].set(w.reshape(-1)[order])
    posf = jnp.zeros((NPAIR,), jnp.int32).at[order].set(pp)
    pos = posf.reshape(T, TOPK)

    x_pad = xbf[src]
    y = _grouped_mlp(block_expert, x_pad, Wg, Wu, Wd, w_pad[:, None])
    return sh + y[pos[:, 0]] + y[pos[:, 1]]
